# baseline (speedup 1.0000x reference)
import functools

import jax
import jax.numpy as jnp
from jax import lax
from jax.experimental import pallas as pl
from jax.experimental.pallas import tpu as pltpu
from jax.experimental.pallas import tpu_sc as plsc

NCLS = 1000
NROWS = 4096 * 26
NW = 32
ROWS_PER_W = NROWS // NW
OUT_PER_W = ROWS_PER_W * NCLS
ZCHUNK = 104000
NZ = OUT_PER_W // ZCHUNK
LANES = 16
IDX_MINOR = 128
NIDX = ROWS_PER_W // IDX_MINOR

_mesh = plsc.VectorSubcoreMesh(core_axis_name="c", subcore_axis_name="s")


@functools.partial(
    pl.kernel,
    out_type=jax.ShapeDtypeStruct((NROWS * NCLS,), jnp.float32),
    mesh=_mesh,
    scratch_types=[
        pltpu.VMEM((ROWS_PER_W,), jnp.int32),
        pltpu.VMEM((ZCHUNK,), jnp.float32),
        pltpu.VMEM((NIDX, IDX_MINOR), jnp.int32),
        pltpu.VMEM((IDX_MINOR,), jnp.float32),
        pltpu.SemaphoreType.DMA,
        pltpu.SemaphoreType.DMA,
    ],
)
def _sc_onehot(x_hbm, out_hbm, idx_v, zbuf, off_v, ones_v, zsem, ssem):
    wid = lax.axis_index("s") * 2 + lax.axis_index("c")
    row0 = wid * ROWS_PER_W
    out0 = row0 * NCLS
    pltpu.sync_copy(x_hbm.at[pl.ds(row0, ROWS_PER_W)], idx_v)

    zeros = jnp.zeros((LANES,), jnp.float32)

    def zfill(i, carry):
        for u in range(8):
            zbuf[pl.ds((i * 8 + u) * LANES, LANES)] = zeros
        return carry

    lax.fori_loop(0, ZCHUNK // (8 * LANES), zfill, 0)
    for k in range(IDX_MINOR // LANES):
        ones_v[pl.ds(k * LANES, LANES)] = jnp.ones((LANES,), jnp.float32)

    def zdma(i, carry):
        pltpu.async_copy(
            zbuf,
            out_hbm.at[pl.ds(out0 + i * ZCHUNK, ZCHUNK)],
            zsem,
        )
        return carry

    lax.fori_loop(0, NZ, zdma, 0)

    lane_iota = lax.iota(jnp.int32, LANES)

    def offs(j, carry):
        for k in range(IDX_MINOR // LANES):
            r = j * IDX_MINOR + k * LANES
            ids = idx_v[pl.ds(r, LANES)]
            off_v[j, pl.ds(k * LANES, LANES)] = (
                (lane_iota + row0 + j * IDX_MINOR + k * LANES) * NCLS + ids
            )
        return carry

    lax.fori_loop(0, NIDX, offs, 0)

    def zwait(i, carry):
        pltpu.make_async_copy(
            zbuf, out_hbm.at[pl.ds(0, ZCHUNK)], zsem
        ).wait()
        return carry

    lax.fori_loop(0, NZ, zwait, 0)

    def sdma(j, carry):
        pltpu.async_copy(ones_v, out_hbm.at[off_v.at[j]], ssem)
        return carry

    lax.fori_loop(0, NIDX, sdma, 0)

    def swait(j, carry):
        pltpu.make_async_copy(
            ones_v, out_hbm.at[pl.ds(0, IDX_MINOR)], ssem
        ).wait()
        return carry

    lax.fori_loop(0, NIDX, swait, 0)


def kernel(x):
    xf = x.reshape(-1).astype(jnp.int32)
    out = _sc_onehot(xf)
    return out.reshape(tuple(x.shape) + (NCLS,))


# speedup vs baseline: 1.5173x; 1.5173x over previous
import jax
import jax.numpy as jnp
from jax import lax
from jax.experimental import pallas as pl
from jax.experimental.pallas import tpu as pltpu

NCLS = 1000
BR = 1024
NBUF = 4


def _onehot_block(x_ref, out_hbm, vbuf, sem):
    i = pl.program_id(0)
    g = pl.num_programs(0)
    b = i % NBUF

    @pl.when(i >= NBUF)
    def _wait_ring():
        pltpu.make_async_copy(
            vbuf.at[b], out_hbm.at[pl.ds((i - NBUF) * BR, BR)], sem.at[b]
        ).wait()

    idx = x_ref[...]
    col = lax.broadcasted_iota(jnp.int32, (BR, NCLS), 1)
    vbuf[b] = (col == idx).astype(jnp.float32)
    pltpu.make_async_copy(
        vbuf.at[b], out_hbm.at[pl.ds(i * BR, BR)], sem.at[b]
    ).start()

    @pl.when(i == g - 1)
    def _drain():
        for d in range(NBUF):
            pltpu.make_async_copy(
                vbuf.at[d], out_hbm.at[pl.ds(0, BR)], sem.at[d]
            ).wait()


def kernel(x):
    xf = x.reshape(-1, 1).astype(jnp.int32)
    n = xf.shape[0]
    out = pl.pallas_call(
        _onehot_block,
        grid=(n // BR,),
        in_specs=[pl.BlockSpec((BR, 1), lambda i: (i, 0))],
        out_specs=pl.BlockSpec(memory_space=pl.ANY),
        out_shape=jax.ShapeDtypeStruct((n, NCLS), jnp.float32),
        scratch_shapes=[
            pltpu.VMEM((NBUF, BR, NCLS), jnp.float32),
            pltpu.SemaphoreType.DMA((NBUF,)),
        ],
        compiler_params=pltpu.CompilerParams(
            dimension_semantics=("arbitrary",),
        ),
    )(xf)
    return out.reshape(tuple(x.shape) + (NCLS,))
